# Initial kernel scaffold; baseline (speedup 1.0000x reference)
import jax
import jax.numpy as jnp
from jax.experimental import pallas as pl


def kernel(atomic_numbers, edge_index, edge_attr, embed_table, Wq, bq, Wk, bk, Wv, bv, We, be, Ws, bs, bn1_g, bn1_b, bn2_g, bn2_b, W1, b1, W2, b2):
    raise NotImplementedError("write your pallas kernel here")


# speedup vs baseline: 3.0738x; 3.0738x over previous
import functools

import jax
import jax.numpy as jnp
import numpy as np
from jax import lax
from jax.experimental import pallas as pl
from jax.experimental.pallas import tpu as pltpu
from jax.experimental.pallas import tpu_sc as plsc

N = 10000
E = 320000
D = 128
L = 3
NC = 2
NS = 16
NW = NC * NS
LANES = 16
NP = 10240
PS = NP // NS
EPW = E // NW
BLK = 80
NBLK = EPW // BLK
GPB = BLK // LANES

_mesh = plsc.VectorSubcoreMesh(core_axis_name="c", subcore_axis_name="s")


def _wid():
    return lax.axis_index("s") * NC + lax.axis_index("c")


@functools.partial(
    pl.kernel,
    out_type=jax.ShapeDtypeStruct((NP, D), jnp.float32),
    mesh=_mesh,
    compiler_params=pltpu.CompilerParams(needs_layout_passes=False),
    scratch_types=[
        pltpu.VMEM((1, BLK), jnp.int32),
        pltpu.VMEM((BLK, D), jnp.float32),
        pltpu.SemaphoreType.DMA,
    ],
)
def _sc_embed(anum_hbm, table_hbm, h_hbm, idxb, rows, sem):
    w = _wid()
    base = w * (NP // NW)

    def body(i, c):
        off = base + i * BLK
        pltpu.sync_copy(anum_hbm.at[pl.ds(off, BLK)], idxb.at[0])
        pltpu.async_copy(table_hbm.at[idxb.at[0]], rows, sem).wait()
        pltpu.sync_copy(rows, h_hbm.at[pl.ds(off, BLK)])
        return c

    lax.fori_loop(0, (NP // NW) // BLK, body, 0)


@functools.partial(
    pl.kernel,
    out_type=(jax.ShapeDtypeStruct((E,), jnp.float32),
              jax.ShapeDtypeStruct((NC, NP), jnp.float32)),
    mesh=_mesh,
    compiler_params=pltpu.CompilerParams(needs_layout_passes=False),
    scratch_types=[
        pltpu.VMEM((1, BLK), jnp.int32),
        pltpu.VMEM((1, BLK), jnp.int32),
        pltpu.VMEM((BLK, D), jnp.float32),
        pltpu.VMEM((BLK, D), jnp.float32),
        pltpu.VMEM((BLK,), jnp.float32),
        pltpu.VMEM((BLK,), jnp.float32),
        pltpu.VMEM((NP,), jnp.float32),
        pltpu.VMEM((PS,), jnp.float32),
        pltpu.VMEM((PS,), jnp.float32),
        pltpu.VMEM_SHARED((NS, NP), jnp.float32),
        pltpu.SemaphoreType.DMA,
        pltpu.SemaphoreType.DMA,
    ],
)
def _sc_alpha(q_hbm, k_hbm, dst_hbm, src_hbm, ea_hbm,
              alpha_hbm, m2_hbm,
              dsti, srci, qrows, krows, eabuf, abuf, mpriv,
              redbuf, redbuf2, mshared, semq, semk):
    sid = lax.axis_index("s")
    cid = lax.axis_index("c")
    ebase = _wid() * EPW
    inv = jnp.float32(1.0 / np.sqrt(D))
    iota = lax.iota(jnp.int32, LANES)

    def initb(i, c):
        mpriv[pl.ds(i * LANES, LANES)] = jnp.full((LANES,), -jnp.inf,
                                                  jnp.float32)
        return c

    lax.fori_loop(0, NP // LANES, initb, 0)

    def blk_body(b, c):
        off = ebase + b * BLK
        pltpu.sync_copy(dst_hbm.at[pl.ds(off, BLK)], dsti.at[0])
        pltpu.sync_copy(src_hbm.at[pl.ds(off, BLK)], srci.at[0])
        pltpu.sync_copy(ea_hbm.at[pl.ds(off, BLK)], eabuf)
        cq = pltpu.async_copy(q_hbm.at[dsti.at[0]], qrows, semq)
        ck = pltpu.async_copy(k_hbm.at[srci.at[0]], krows, semk)
        cq.wait()
        ck.wait()
        for g in range(GPB):
            rows = iota + g * LANES

            def dot_body(j, acc):
                for u in range(8):
                    col = jnp.full((LANES,), 0, jnp.int32) + (j * 8 + u)
                    qc = plsc.load_gather(qrows, [rows, col])
                    kc = plsc.load_gather(krows, [rows, col])
                    acc = acc + qc * kc
                return acc

            acc = lax.fori_loop(0, D // 8, dot_body,
                                jnp.zeros((LANES,), jnp.float32))
            ss = pl.ds(g * LANES, LANES)
            ag = acc * inv + eabuf[ss]
            abuf[ss] = ag
            dg = dsti[0, ss]
            cur = plsc.load_gather(mpriv, [dg])
            act = jnp.where(ag > cur, jnp.int32(1), jnp.int32(0))

            def rcond(a):
                return jnp.max(a) > 0

            def rbody(a):
                msk = a > 0
                plsc.store_scatter(mpriv, [dg], ag, mask=msk)
                c2 = plsc.load_gather(mpriv, [dg])
                return jnp.where(msk & (ag > c2), jnp.int32(1), jnp.int32(0))

            lax.while_loop(rcond, rbody, act)
        pltpu.sync_copy(abuf, alpha_hbm.at[pl.ds(off, BLK)])
        return c

    lax.fori_loop(0, NBLK, blk_body, 0)

    pltpu.sync_copy(mpriv, mshared.at[sid])
    plsc.subcore_barrier()
    nbase = sid * PS
    pltpu.sync_copy(mshared.at[0, pl.ds(nbase, PS)], redbuf)

    def tred(t, c):
        pltpu.sync_copy(mshared.at[t, pl.ds(nbase, PS)], redbuf2)

        def mx(i, c2):
            s = pl.ds(i * LANES, LANES)
            redbuf[s] = jnp.maximum(redbuf[s], redbuf2[s])
            return c2

        lax.fori_loop(0, PS // LANES, mx, 0)
        return c

    lax.fori_loop(1, NS, tred, 0)
    pltpu.sync_copy(redbuf, m2_hbm.at[cid, pl.ds(nbase, PS)])


@functools.partial(
    pl.kernel,
    out_type=(jax.ShapeDtypeStruct((NC, NP, D), jnp.float32),
              jax.ShapeDtypeStruct((NC, NP), jnp.float32)),
    mesh=_mesh,
    compiler_params=pltpu.CompilerParams(needs_layout_passes=False),
    scratch_types=[
        pltpu.VMEM((1, BLK), jnp.int32),
        pltpu.VMEM((1, BLK), jnp.int32),
        pltpu.VMEM((BLK, D), jnp.float32),
        pltpu.VMEM((BLK,), jnp.float32),
        pltpu.VMEM((BLK,), jnp.float32),
        pltpu.VMEM((NP,), jnp.float32),
        pltpu.VMEM((NP,), jnp.float32),
        pltpu.VMEM((PS,), jnp.float32),
        pltpu.VMEM((PS,), jnp.float32),
        pltpu.VMEM_SHARED((NP, D), jnp.float32),
        pltpu.VMEM_SHARED((NS, NP), jnp.float32),
        pltpu.SemaphoreType.DMA,
    ],
)
def _sc_aggregate(v_hbm, alpha_hbm, m2_hbm, dst_hbm, src_hbm,
                  out2_hbm, den2_hbm,
                  dsti, srci, vrows, abuf, wbuf, mloc, dpriv,
                  redbuf, redbuf2, oshared, dshared, semv):
    sid = lax.axis_index("s")
    cid = lax.axis_index("c")
    ebase = _wid() * EPW

    def mker(i, c):
        s = pl.ds(i * PS, PS)
        pltpu.sync_copy(m2_hbm.at[0, s], redbuf)
        pltpu.sync_copy(m2_hbm.at[1, s], redbuf2)

        def mx(j, c2):
            ss = pl.ds(j * LANES, LANES)
            mloc[pl.ds(i * PS + j * LANES, LANES)] = jnp.maximum(
                redbuf[ss], redbuf2[ss])
            return c2

        lax.fori_loop(0, PS // LANES, mx, 0)
        return c

    lax.fori_loop(0, NP // PS, mker, 0)

    def zd(i, c):
        dpriv[pl.ds(i * LANES, LANES)] = jnp.zeros((LANES,), jnp.float32)
        return c

    lax.fori_loop(0, NP // LANES, zd, 0)

    def zv(r, c):
        for j in range(D // LANES):
            vrows[r, pl.ds(j * LANES, LANES)] = jnp.zeros((LANES,),
                                                          jnp.float32)
        return c

    lax.fori_loop(0, BLK, zv, 0)

    def zo(i, c):
        pltpu.sync_copy(vrows, oshared.at[pl.ds(sid * PS + i * BLK, BLK)])
        return c

    lax.fori_loop(0, PS // BLK, zo, 0)
    plsc.subcore_barrier()

    def blk_body(b, c):
        off = ebase + b * BLK
        pltpu.sync_copy(dst_hbm.at[pl.ds(off, BLK)], dsti.at[0])
        pltpu.sync_copy(src_hbm.at[pl.ds(off, BLK)], srci.at[0])
        pltpu.sync_copy(alpha_hbm.at[pl.ds(off, BLK)], abuf)
        pltpu.async_copy(v_hbm.at[srci.at[0]], vrows, semv).wait()
        for g in range(GPB):
            ss = pl.ds(g * LANES, LANES)
            dg = dsti[0, ss]
            mg = plsc.load_gather(mloc, [dg])
            ag = jnp.exp(abuf[ss] - mg)
            wbuf[ss] = ag
            plsc.addupdate_scatter(dpriv, [dg], ag)

        def scale(e, c2):
            asp = plsc.load_gather(wbuf,
                                   [jnp.full((LANES,), 0, jnp.int32) + e])
            for j in range(D // LANES):
                s2 = pl.ds(j * LANES, LANES)
                vrows[e, s2] = vrows[e, s2] * asp
            return c2

        lax.fori_loop(0, BLK, scale, 0)
        pltpu.sync_copy(vrows, oshared.at[dsti.at[0]], add=True)
        return c

    lax.fori_loop(0, NBLK, blk_body, 0)

    pltpu.sync_copy(dpriv, dshared.at[sid])
    plsc.subcore_barrier()
    nbase = sid * PS
    pltpu.sync_copy(dshared.at[0, pl.ds(nbase, PS)], redbuf)

    def tred(t, c):
        pltpu.sync_copy(dshared.at[t, pl.ds(nbase, PS)], redbuf2)

        def ad(i, c2):
            s = pl.ds(i * LANES, LANES)
            redbuf[s] = redbuf[s] + redbuf2[s]
            return c2

        lax.fori_loop(0, PS // LANES, ad, 0)
        return c

    lax.fori_loop(1, NS, tred, 0)
    pltpu.sync_copy(redbuf, den2_hbm.at[cid, pl.ds(nbase, PS)])
    pltpu.sync_copy(oshared.at[pl.ds(nbase, PS)],
                    out2_hbm.at[cid, pl.ds(nbase, PS)])


BS = 400
GR = N // BS
EB = E // GR


def _tc_pre_body(x_ref, w_ref, b_ref, ea_ref, wb_ref,
                 q_ref, k_ref, v_ref, s_ref, eat_ref):
    y = jnp.dot(x_ref[...], w_ref[...],
                preferred_element_type=jnp.float32) + b_ref[...]
    q_ref[...] = y[:, :D]
    k_ref[...] = y[:, D:2 * D]
    v_ref[...] = y[:, 2 * D:3 * D]
    s_ref[...] = y[:, 3 * D:]
    eat_ref[...] = ea_ref[...] * wb_ref[0, 0] + wb_ref[0, 1]


def _eb_spec():
    return pl.BlockSpec((1, 1, EB), lambda i: (i, 0, 0))


def _tc_pre(h, wcat, bcat, ea2d, wb):
    return pl.pallas_call(
        _tc_pre_body,
        grid=(GR,),
        in_specs=[pl.BlockSpec((BS, D), lambda i: (i, 0)),
                  pl.BlockSpec((D, 4 * D), lambda i: (0, 0)),
                  pl.BlockSpec((1, 4 * D), lambda i: (0, 0)),
                  _eb_spec(),
                  pl.BlockSpec((1, 128), lambda i: (0, 0))],
        out_specs=[pl.BlockSpec((BS, D), lambda i: (i, 0))] * 4 +
                  [_eb_spec()],
        out_shape=[jax.ShapeDtypeStruct((N, D), jnp.float32)] * 4 +
                  [jax.ShapeDtypeStruct((GR, 1, EB), jnp.float32)],
    )(h, wcat, bcat, ea2d, wb)


def _tc_post_body(o2_ref, d2_ref, s_ref, g1_ref, b1_ref, g2_ref, b2_ref,
                  h_ref):
    o = o2_ref[0, :N, :] + o2_ref[1, :N, :]
    den = d2_ref[0, :N, :] + d2_ref[1, :N, :]
    x = o / (den + 1e-16) + s_ref[...]
    mu = jnp.mean(x, axis=0, keepdims=True)
    xc = x - mu
    var = jnp.mean(xc * xc, axis=0, keepdims=True)
    x = xc / jnp.sqrt(var + 1e-5) * g1_ref[...] + b1_ref[...]
    mu = jnp.mean(x, axis=0, keepdims=True)
    xc = x - mu
    var = jnp.mean(xc * xc, axis=0, keepdims=True)
    x = xc / jnp.sqrt(var + 1e-5) * g2_ref[...] + b2_ref[...]
    h_ref[...] = jax.nn.softplus(x)


def _tc_post(out2, den2, s, g1, b1, g2, b2):
    return pl.pallas_call(
        _tc_post_body,
        out_shape=jax.ShapeDtypeStruct((N, D), jnp.float32),
    )(out2, den2, s, g1, b1, g2, b2)


def _tc_final_body(h_ref, w1t_ref, b1_ref, w2_ref, b2_ref, o_ref):
    pooled = jnp.mean(h_ref[...], axis=0, keepdims=True)
    hid = jax.nn.softplus(
        jnp.dot(pooled, w1t_ref[...],
                preferred_element_type=jnp.float32) + b1_ref[...])
    pred = jnp.sum(hid * w2_ref[...]) + b2_ref[0, 0]
    o_ref[...] = jnp.zeros((8, 128), jnp.float32) + pred


def _tc_final(h, w1t, b1, w2, b2):
    return pl.pallas_call(
        _tc_final_body,
        out_shape=jax.ShapeDtypeStruct((8, 128), jnp.float32),
    )(h, w1t, b1, w2, b2)


def kernel(atomic_numbers, edge_index, edge_attr, embed_table, Wq, bq, Wk,
           bk, Wv, bv, We, be, Ws, bs, bn1_g, bn1_b, bn2_g, bn2_b, W1, b1,
           W2, b2):
    anum = jnp.pad(atomic_numbers.astype(jnp.int32), (0, NP - N))
    src = edge_index[0].astype(jnp.int32)
    dst = edge_index[1].astype(jnp.int32)
    ea2d = edge_attr[:, 0].astype(jnp.float32).reshape(GR, 1, EB)
    h = _sc_embed(anum, embed_table)[:N]
    for l in range(L):
        wcat = jnp.concatenate([Wq[l], Wk[l], Wv[l], Ws[l]], axis=0).T
        bcat = jnp.concatenate([bq[l], bk[l], bv[l], bs[l]]).reshape(
            1, 4 * D)
        wb = jnp.zeros((1, 128), jnp.float32).at[0, 0].set(
            We[l, 0, 0]).at[0, 1].set(be[l, 0])
        q, k, v, s, eat = _tc_pre(h, wcat, bcat, ea2d, wb)
        alpha, m2 = _sc_alpha(q, k, dst, src, eat.reshape(E))
        out2, den2 = _sc_aggregate(v, alpha, m2, dst, src)
        h = _tc_post(out2, den2.reshape(NC, NP, 1), s,
                     bn1_g[l].reshape(1, D), bn1_b[l].reshape(1, D),
                     bn2_g[l].reshape(1, D), bn2_b[l].reshape(1, D))
    out = _tc_final(h, W1.T, b1.reshape(1, 2 * D), W2, b2.reshape(1, 1))
    return out[0, :1]


# speedup vs baseline: 4.0974x; 1.3330x over previous
import functools

import jax
import jax.numpy as jnp
import numpy as np
from jax import lax
from jax.experimental import pallas as pl
from jax.experimental.pallas import tpu as pltpu
from jax.experimental.pallas import tpu_sc as plsc

N = 10000
E = 320000
D = 128
L = 3
NC = 2
NS = 16
NW = NC * NS
LANES = 16
NP = 10240
PS = NP // NS
EPW = E // NW
BLK = 80
NBLK = EPW // BLK
GPB = BLK // LANES

_mesh = plsc.VectorSubcoreMesh(core_axis_name="c", subcore_axis_name="s")


def _wid():
    return lax.axis_index("s") * NC + lax.axis_index("c")


@functools.partial(
    pl.kernel,
    out_type=jax.ShapeDtypeStruct((NP, D), jnp.float32),
    mesh=_mesh,
    compiler_params=pltpu.CompilerParams(needs_layout_passes=False),
    scratch_types=[
        pltpu.VMEM((1, BLK), jnp.int32),
        pltpu.VMEM((BLK, D), jnp.float32),
        pltpu.SemaphoreType.DMA,
    ],
)
def _sc_embed(anum_hbm, table_hbm, h_hbm, idxb, rows, sem):
    w = _wid()
    base = w * (NP // NW)

    def body(i, c):
        off = base + i * BLK
        pltpu.sync_copy(anum_hbm.at[pl.ds(off, BLK)], idxb.at[0])
        pltpu.async_copy(table_hbm.at[idxb.at[0]], rows, sem).wait()
        pltpu.sync_copy(rows, h_hbm.at[pl.ds(off, BLK)])
        return c

    lax.fori_loop(0, (NP // NW) // BLK, body, 0)


@functools.partial(
    pl.kernel,
    out_type=(jax.ShapeDtypeStruct((NW, NBLK, BLK), jnp.float32),
              jax.ShapeDtypeStruct((NC, NP), jnp.float32)),
    mesh=_mesh,
    compiler_params=pltpu.CompilerParams(needs_layout_passes=False),
    scratch_types=[
        pltpu.VMEM((NBLK, BLK), jnp.int32),
        pltpu.VMEM((NBLK, BLK), jnp.int32),
        pltpu.VMEM((NBLK, BLK), jnp.float32),
        pltpu.VMEM((NBLK, BLK), jnp.float32),
        pltpu.VMEM((2, BLK, D), jnp.float32),
        pltpu.VMEM((2, BLK, D), jnp.float32),
        pltpu.VMEM((NP,), jnp.float32),
        pltpu.VMEM((PS,), jnp.float32),
        pltpu.VMEM((PS,), jnp.float32),
        pltpu.VMEM_SHARED((NS, NP), jnp.float32),
        pltpu.SemaphoreType.DMA,
        pltpu.SemaphoreType.DMA,
        pltpu.SemaphoreType.DMA,
        pltpu.SemaphoreType.DMA,
    ],
)
def _sc_alpha(q_hbm, k_hbm, dst_hbm, src_hbm, ea_hbm,
              alpha_hbm, m2_hbm,
              dsti, srci, eatv, abuf, qrows, krows, mpriv,
              redbuf, redbuf2, mshared, sq0, sq1, sk0, sk1):
    sid = lax.axis_index("s")
    cid = lax.axis_index("c")
    w = _wid()
    inv = jnp.float32(1.0 / np.sqrt(D))
    iota = lax.iota(jnp.int32, LANES)
    sems = ((sq0, sk0), (sq1, sk1))

    pltpu.sync_copy(dst_hbm.at[w], dsti)
    pltpu.sync_copy(src_hbm.at[w], srci)
    pltpu.sync_copy(ea_hbm.at[w], eatv)

    def start(b, par):
        pltpu.async_copy(q_hbm.at[dsti.at[b]], qrows.at[par], sems[par][0])
        pltpu.async_copy(k_hbm.at[srci.at[b]], krows.at[par], sems[par][1])

    def wait(par):
        pltpu.make_async_copy(q_hbm.at[dsti.at[0]], qrows.at[par],
                              sems[par][0]).wait()
        pltpu.make_async_copy(k_hbm.at[srci.at[0]], krows.at[par],
                              sems[par][1]).wait()

    start(0, 0)
    start(1, 1)

    def initb(i, c):
        mpriv[pl.ds(i * LANES, LANES)] = jnp.full((LANES,), -jnp.inf,
                                                  jnp.float32)
        return c

    lax.fori_loop(0, NP // LANES, initb, 0)

    def compute(b, par):
        qr = qrows.at[par]
        kr = krows.at[par]
        for g in range(GPB):
            rows = iota + g * LANES

            def dot_body(j, acc):
                for u in range(8):
                    col = jnp.full((LANES,), 0, jnp.int32) + (j * 8 + u)
                    acc = acc + (plsc.load_gather(qr, [rows, col]) *
                                 plsc.load_gather(kr, [rows, col]))
                return acc

            acc = lax.fori_loop(0, D // 8, dot_body,
                                jnp.zeros((LANES,), jnp.float32))
            ss = pl.ds(g * LANES, LANES)
            ag = acc * inv + eatv[b, ss]
            abuf[b, ss] = ag
            dg = dsti[b, ss]
            cur = plsc.load_gather(mpriv, [dg])
            act = jnp.where(ag > cur, jnp.int32(1), jnp.int32(0))

            def rcond(a):
                return jnp.max(a) > 0

            def rbody(a):
                msk = a > 0
                plsc.store_scatter(mpriv, [dg], ag, mask=msk)
                c2 = plsc.load_gather(mpriv, [dg])
                return jnp.where(msk & (ag > c2), jnp.int32(1),
                                 jnp.int32(0))

            lax.while_loop(rcond, rbody, act)

    def pair(s, c):
        for par in range(2):
            b = 2 * s + par
            wait(par)
            compute(b, par)

            @pl.when(b + 2 < NBLK)
            def _():
                start(b + 2, par)
        return c

    lax.fori_loop(0, (NBLK - 1) // 2, pair, 0)
    wait(0)
    compute(NBLK - 1, 0)
    pltpu.sync_copy(abuf, alpha_hbm.at[w])

    pltpu.sync_copy(mpriv, mshared.at[sid])
    plsc.subcore_barrier()
    nbase = sid * PS
    pltpu.sync_copy(mshared.at[0, pl.ds(nbase, PS)], redbuf)

    def tred(t, c):
        pltpu.sync_copy(mshared.at[t, pl.ds(nbase, PS)], redbuf2)

        def mx(i, c2):
            s = pl.ds(i * LANES, LANES)
            redbuf[s] = jnp.maximum(redbuf[s], redbuf2[s])
            return c2

        lax.fori_loop(0, PS // LANES, mx, 0)
        return c

    lax.fori_loop(1, NS, tred, 0)
    pltpu.sync_copy(redbuf, m2_hbm.at[cid, pl.ds(nbase, PS)])


@functools.partial(
    pl.kernel,
    out_type=(jax.ShapeDtypeStruct((NC, NP, D), jnp.float32),
              jax.ShapeDtypeStruct((NC, NP // 128, 128), jnp.float32)),
    mesh=_mesh,
    compiler_params=pltpu.CompilerParams(needs_layout_passes=False),
    scratch_types=[
        pltpu.VMEM((2, 1, BLK), jnp.int32),
        pltpu.VMEM((2, 1, BLK), jnp.int32),
        pltpu.VMEM((2, 1, BLK), jnp.float32),
        pltpu.VMEM((2, BLK, D), jnp.float32),
        pltpu.VMEM((BLK,), jnp.float32),
        pltpu.VMEM((NP,), jnp.float32),
        pltpu.VMEM((NP // 128, 128), jnp.float32),
        pltpu.VMEM((1, NP // 128), jnp.int32),
        pltpu.VMEM((PS,), jnp.float32),
        pltpu.VMEM((PS,), jnp.float32),
        pltpu.VMEM_SHARED((NP, D), jnp.float32),
        pltpu.VMEM_SHARED((NP // 128, 128), jnp.float32),
        pltpu.SemaphoreType.DMA,
        pltpu.SemaphoreType.DMA,
        pltpu.SemaphoreType.DMA,
        pltpu.SemaphoreType.DMA,
    ],
)
def _sc_aggregate(v_hbm, alpha_hbm, m2_hbm, dst_hbm, src_hbm,
                  out2_hbm, den2_hbm,
                  dsti, srci, abufb, vrows, wbuf, mloc, dpriv, idx80,
                  redbuf, redbuf2, oshared, dshared, i0, i1, g0, g1):
    sid = lax.axis_index("s")
    cid = lax.axis_index("c")
    w = _wid()
    isem = (i0, i1)
    gsem = (g0, g1)

    def istart(b, par):
        pltpu.async_copy(dst_hbm.at[w, pl.ds(b, 1)], dsti.at[par],
                         isem[par])
        pltpu.async_copy(src_hbm.at[w, pl.ds(b, 1)], srci.at[par],
                         isem[par])
        pltpu.async_copy(alpha_hbm.at[w, pl.ds(b, 1)], abufb.at[par],
                         isem[par])

    def iwait(par):
        pltpu.make_async_copy(dst_hbm.at[w, pl.ds(0, 1)], dsti.at[par],
                              isem[par]).wait()
        pltpu.make_async_copy(src_hbm.at[w, pl.ds(0, 1)], srci.at[par],
                              isem[par]).wait()
        pltpu.make_async_copy(alpha_hbm.at[w, pl.ds(0, 1)], abufb.at[par],
                              isem[par]).wait()

    def gstart(b, par):
        pltpu.async_copy(v_hbm.at[srci.at[par, 0]], vrows.at[par],
                         gsem[par])

    def gwait(par):
        pltpu.make_async_copy(v_hbm.at[srci.at[0, 0]], vrows.at[par],
                              gsem[par]).wait()

    istart(0, 0)
    istart(1, 1)

    def mker(i, c):
        s = pl.ds(i * PS, PS)
        pltpu.sync_copy(m2_hbm.at[0, s], redbuf)
        pltpu.sync_copy(m2_hbm.at[1, s], redbuf2)

        def mx(j, c2):
            ss = pl.ds(j * LANES, LANES)
            mloc[pl.ds(i * PS + j * LANES, LANES)] = jnp.maximum(
                redbuf[ss], redbuf2[ss])
            return c2

        lax.fori_loop(0, PS // LANES, mx, 0)
        return c

    lax.fori_loop(0, NP // PS, mker, 0)

    iota = lax.iota(jnp.int32, LANES)

    def zd(r, c):
        for j in range(D // LANES):
            dpriv[r, pl.ds(j * LANES, LANES)] = jnp.zeros((LANES,),
                                                          jnp.float32)
        return c

    lax.fori_loop(0, NP // 128, zd, 0)
    for j in range(NP // 128 // LANES):
        idx80[0, pl.ds(j * LANES, LANES)] = iota + j * LANES

    def zv(r, c):
        for j in range(D // LANES):
            vrows[0, r, pl.ds(j * LANES, LANES)] = jnp.zeros(
                (LANES,), jnp.float32)
        return c

    lax.fori_loop(0, BLK, zv, 0)

    def zo(i, c):
        pltpu.sync_copy(vrows.at[0],
                        oshared.at[pl.ds(sid * PS + i * BLK, BLK)])
        return c

    lax.fori_loop(0, PS // BLK, zo, 0)
    @pl.when(sid < NP // 128 // 8)
    def _():
        pltpu.sync_copy(vrows.at[0, pl.ds(0, 8)],
                        dshared.at[pl.ds(sid * 8, 8)])

    plsc.subcore_barrier()
    iwait(0)
    gstart(0, 0)

    def blk(b, par):
        @pl.when(b + 1 < NBLK)
        def _():
            iwait(1 - par)
            gstart(b + 1, 1 - par)

        gwait(par)
        for g in range(GPB):
            ss = pl.ds(g * LANES, LANES)
            dg = dsti[par, 0, ss]
            mg = plsc.load_gather(mloc, [dg])
            ag = jnp.exp(abufb[par, 0, ss] - mg)
            wbuf[ss] = ag
            dr = lax.shift_right_logical(dg, 7)
            dc = jnp.bitwise_and(dg, 127)
            plsc.addupdate_scatter(dpriv, [dr, dc], ag)

        def scale(e, c2):
            asp = plsc.load_gather(wbuf,
                                   [jnp.full((LANES,), 0, jnp.int32) + e])
            for j in range(D // LANES):
                s2 = pl.ds(j * LANES, LANES)
                vrows[par, e, s2] = vrows[par, e, s2] * asp
            return c2

        lax.fori_loop(0, BLK, scale, 0)
        pltpu.sync_copy(vrows.at[par], oshared.at[dsti.at[par, 0]],
                        add=True)

        @pl.when(b + 2 < NBLK)
        def _():
            istart(b + 2, par)

    def pair(s, c):
        blk(2 * s, 0)
        blk(2 * s + 1, 1)
        return c

    lax.fori_loop(0, (NBLK - 1) // 2, pair, 0)
    blk(NBLK - 1, 0)

    pltpu.sync_copy(dpriv, dshared.at[idx80.at[0]], add=True)
    plsc.subcore_barrier()
    nbase = sid * PS

    @pl.when(sid < NP // 128 // 8)
    def _():
        pltpu.sync_copy(dshared.at[pl.ds(sid * 8, 8)],
                        den2_hbm.at[cid, pl.ds(sid * 8, 8)])

    pltpu.sync_copy(oshared.at[pl.ds(nbase, PS)],
                    out2_hbm.at[cid, pl.ds(nbase, PS)])


BS = 400
GR = N // BS
EB = E // GR


def _tc_pre_body(x_ref, w_ref, b_ref, ea_ref, wb_ref,
                 q_ref, k_ref, v_ref, s_ref, eat_ref):
    y = jnp.dot(x_ref[...], w_ref[...],
                preferred_element_type=jnp.float32) + b_ref[...]
    q_ref[...] = y[:, :D]
    k_ref[...] = y[:, D:2 * D]
    v_ref[...] = y[:, 2 * D:3 * D]
    s_ref[...] = y[:, 3 * D:]
    eat_ref[...] = ea_ref[...] * wb_ref[0, 0] + wb_ref[0, 1]


def _eb_spec():
    return pl.BlockSpec((1, 1, EB), lambda i: (i, 0, 0))


def _tc_pre(h, wcat, bcat, ea2d, wb):
    return pl.pallas_call(
        _tc_pre_body,
        grid=(GR,),
        in_specs=[pl.BlockSpec((BS, D), lambda i: (i, 0)),
                  pl.BlockSpec((D, 4 * D), lambda i: (0, 0)),
                  pl.BlockSpec((1, 4 * D), lambda i: (0, 0)),
                  _eb_spec(),
                  pl.BlockSpec((1, 128), lambda i: (0, 0))],
        out_specs=[pl.BlockSpec((BS, D), lambda i: (i, 0))] * 4 +
                  [_eb_spec()],
        out_shape=[jax.ShapeDtypeStruct((N, D), jnp.float32)] * 4 +
                  [jax.ShapeDtypeStruct((GR, 1, EB), jnp.float32)],
    )(h, wcat, bcat, ea2d, wb)


def _tc_post_body(o2_ref, d2_ref, s_ref, g1_ref, b1_ref, g2_ref, b2_ref,
                  h_ref):
    o = o2_ref[0, :N, :] + o2_ref[1, :N, :]
    den = d2_ref[0, :N, :] + d2_ref[1, :N, :]
    x = o / (den + 1e-16) + s_ref[...]
    mu = jnp.mean(x, axis=0, keepdims=True)
    xc = x - mu
    var = jnp.mean(xc * xc, axis=0, keepdims=True)
    x = xc / jnp.sqrt(var + 1e-5) * g1_ref[...] + b1_ref[...]
    mu = jnp.mean(x, axis=0, keepdims=True)
    xc = x - mu
    var = jnp.mean(xc * xc, axis=0, keepdims=True)
    x = xc / jnp.sqrt(var + 1e-5) * g2_ref[...] + b2_ref[...]
    h_ref[...] = jax.nn.softplus(x)


def _tc_post(out2, den2, s, g1, b1, g2, b2):
    return pl.pallas_call(
        _tc_post_body,
        out_shape=jax.ShapeDtypeStruct((N, D), jnp.float32),
    )(out2, den2, s, g1, b1, g2, b2)


def _tc_final_body(h_ref, w1t_ref, b1_ref, w2_ref, b2_ref, o_ref):
    pooled = jnp.mean(h_ref[...], axis=0, keepdims=True)
    hid = jax.nn.softplus(
        jnp.dot(pooled, w1t_ref[...],
                preferred_element_type=jnp.float32) + b1_ref[...])
    pred = jnp.sum(hid * w2_ref[...]) + b2_ref[0, 0]
    o_ref[...] = jnp.zeros((8, 128), jnp.float32) + pred


def _tc_final(h, w1t, b1, w2, b2):
    return pl.pallas_call(
        _tc_final_body,
        out_shape=jax.ShapeDtypeStruct((8, 128), jnp.float32),
    )(h, w1t, b1, w2, b2)


def kernel(atomic_numbers, edge_index, edge_attr, embed_table, Wq, bq, Wk,
           bk, Wv, bv, We, be, Ws, bs, bn1_g, bn1_b, bn2_g, bn2_b, W1, b1,
           W2, b2):
    anum = jnp.pad(atomic_numbers.astype(jnp.int32), (0, NP - N))
    src = edge_index[0].astype(jnp.int32).reshape(NW, NBLK, BLK)
    dst = edge_index[1].astype(jnp.int32).reshape(NW, NBLK, BLK)
    ea2d = edge_attr[:, 0].astype(jnp.float32).reshape(GR, 1, EB)
    h = _sc_embed(anum, embed_table)[:N]
    for l in range(L):
        wcat = jnp.concatenate([Wq[l], Wk[l], Wv[l], Ws[l]], axis=0).T
        bcat = jnp.concatenate([bq[l], bk[l], bv[l], bs[l]]).reshape(
            1, 4 * D)
        wb = jnp.zeros((1, 128), jnp.float32).at[0, 0].set(
            We[l, 0, 0]).at[0, 1].set(be[l, 0])
        q, k, v, s, eat = _tc_pre(h, wcat, bcat, ea2d, wb)
        alpha, m2 = _sc_alpha(q, k, dst, src, eat.reshape(NW, NBLK, BLK))
        out2, den2 = _sc_aggregate(v, alpha, m2, dst, src)
        h = _tc_post(out2, den2.reshape(NC, NP, 1), s,
                     bn1_g[l].reshape(1, D), bn1_b[l].reshape(1, D),
                     bn2_g[l].reshape(1, D), bn2_b[l].reshape(1, D))
    out = _tc_final(h, W1.T, b1.reshape(1, 2 * D), W2, b2.reshape(1, 1))
    return out[0, :1]


# speedup vs baseline: 4.1048x; 1.0018x over previous
import functools

import jax
import jax.numpy as jnp
import numpy as np
from jax import lax
from jax.experimental import pallas as pl
from jax.experimental.pallas import tpu as pltpu
from jax.experimental.pallas import tpu_sc as plsc

N = 10000
E = 320000
D = 128
L = 3
NC = 2
NS = 16
NW = NC * NS
LANES = 16
NP = 10240
PS = NP // NS
EPW = E // NW
BLK = 80
NBLK = EPW // BLK
GPB = BLK // LANES

_mesh = plsc.VectorSubcoreMesh(core_axis_name="c", subcore_axis_name="s")


def _wid():
    return lax.axis_index("s") * NC + lax.axis_index("c")


@functools.partial(
    pl.kernel,
    out_type=jax.ShapeDtypeStruct((NP, D), jnp.float32),
    mesh=_mesh,
    compiler_params=pltpu.CompilerParams(needs_layout_passes=False),
    scratch_types=[
        pltpu.VMEM((1, BLK), jnp.int32),
        pltpu.VMEM((BLK, D), jnp.float32),
        pltpu.SemaphoreType.DMA,
    ],
)
def _sc_embed(anum_hbm, table_hbm, h_hbm, idxb, rows, sem):
    w = _wid()
    base = w * (NP // NW)

    def body(i, c):
        off = base + i * BLK
        pltpu.sync_copy(anum_hbm.at[pl.ds(off, BLK)], idxb.at[0])
        pltpu.async_copy(table_hbm.at[idxb.at[0]], rows, sem).wait()
        pltpu.sync_copy(rows, h_hbm.at[pl.ds(off, BLK)])
        return c

    lax.fori_loop(0, (NP // NW) // BLK, body, 0)


@functools.partial(
    pl.kernel,
    out_type=(jax.ShapeDtypeStruct((NW, NBLK, BLK), jnp.float32),
              jax.ShapeDtypeStruct((NC, NP), jnp.float32)),
    mesh=_mesh,
    compiler_params=pltpu.CompilerParams(needs_layout_passes=False),
    scratch_types=[
        pltpu.VMEM((NBLK, BLK), jnp.int32),
        pltpu.VMEM((NBLK, BLK), jnp.int32),
        pltpu.VMEM((NBLK, BLK), jnp.float32),
        pltpu.VMEM((NBLK, BLK), jnp.float32),
        pltpu.VMEM((2, BLK, D), jnp.float32),
        pltpu.VMEM((2, BLK, D), jnp.float32),
        pltpu.VMEM((NP,), jnp.float32),
        pltpu.VMEM((PS,), jnp.float32),
        pltpu.VMEM((PS,), jnp.float32),
        pltpu.VMEM_SHARED((NS, NP), jnp.float32),
        pltpu.SemaphoreType.DMA,
        pltpu.SemaphoreType.DMA,
        pltpu.SemaphoreType.DMA,
        pltpu.SemaphoreType.DMA,
    ],
)
def _sc_alpha(q_hbm, k_hbm, dst_hbm, src_hbm, ea_hbm,
              alpha_hbm, m2_hbm,
              dsti, srci, eatv, abuf, qrows, krows, mpriv,
              redbuf, redbuf2, mshared, sq0, sq1, sk0, sk1):
    sid = lax.axis_index("s")
    cid = lax.axis_index("c")
    w = _wid()
    inv = jnp.float32(1.0 / np.sqrt(D))
    iota = lax.iota(jnp.int32, LANES)
    sems = ((sq0, sk0), (sq1, sk1))

    pltpu.sync_copy(dst_hbm.at[w], dsti)
    pltpu.sync_copy(src_hbm.at[w], srci)
    pltpu.sync_copy(ea_hbm.at[w], eatv)

    def start(b, par):
        pltpu.async_copy(q_hbm.at[dsti.at[b]], qrows.at[par], sems[par][0])
        pltpu.async_copy(k_hbm.at[srci.at[b]], krows.at[par], sems[par][1])

    def wait(par):
        pltpu.make_async_copy(q_hbm.at[dsti.at[0]], qrows.at[par],
                              sems[par][0]).wait()
        pltpu.make_async_copy(k_hbm.at[srci.at[0]], krows.at[par],
                              sems[par][1]).wait()

    start(0, 0)
    start(1, 1)

    def initb(i, c):
        mpriv[pl.ds(i * LANES, LANES)] = jnp.full((LANES,), -jnp.inf,
                                                  jnp.float32)
        return c

    lax.fori_loop(0, NP // LANES, initb, 0)

    zvec = jnp.zeros((LANES,), jnp.int32)

    def compute(b, par):
        qr = qrows.at[par]
        kr = krows.at[par]
        for g in range(GPB):
            rbase = (iota + g * LANES) * D

            def dot_body(j, carry):
                acc, idx = carry
                for u in range(8):
                    iu = idx if u == 0 else idx + u
                    acc = acc + (plsc.load_gather(qr, [zvec, iu]) *
                                 plsc.load_gather(kr, [zvec, iu]))
                return (acc, idx + 8)

            acc, _ = lax.fori_loop(
                0, D // 8, dot_body,
                (jnp.zeros((LANES,), jnp.float32), rbase))
            ss = pl.ds(g * LANES, LANES)
            ag = acc * inv + eatv[b, ss]
            abuf[b, ss] = ag
            dg = dsti[b, ss]
            cur = plsc.load_gather(mpriv, [dg])
            act = jnp.where(ag > cur, jnp.int32(1), jnp.int32(0))

            def rcond(a):
                return jnp.max(a) > 0

            def rbody(a):
                msk = a > 0
                plsc.store_scatter(mpriv, [dg], ag, mask=msk)
                c2 = plsc.load_gather(mpriv, [dg])
                return jnp.where(msk & (ag > c2), jnp.int32(1),
                                 jnp.int32(0))

            lax.while_loop(rcond, rbody, act)

    def pair(s, c):
        for par in range(2):
            b = 2 * s + par
            wait(par)
            compute(b, par)

            @pl.when(b + 2 < NBLK)
            def _():
                start(b + 2, par)
        return c

    lax.fori_loop(0, (NBLK - 1) // 2, pair, 0)
    wait(0)
    compute(NBLK - 1, 0)
    pltpu.sync_copy(abuf, alpha_hbm.at[w])

    pltpu.sync_copy(mpriv, mshared.at[sid])
    plsc.subcore_barrier()
    nbase = sid * PS
    pltpu.sync_copy(mshared.at[0, pl.ds(nbase, PS)], redbuf)

    def tred(t, c):
        pltpu.sync_copy(mshared.at[t, pl.ds(nbase, PS)], redbuf2)

        def mx(i, c2):
            s = pl.ds(i * LANES, LANES)
            redbuf[s] = jnp.maximum(redbuf[s], redbuf2[s])
            return c2

        lax.fori_loop(0, PS // LANES, mx, 0)
        return c

    lax.fori_loop(1, NS, tred, 0)
    pltpu.sync_copy(redbuf, m2_hbm.at[cid, pl.ds(nbase, PS)])


@functools.partial(
    pl.kernel,
    out_type=(jax.ShapeDtypeStruct((NC, NP, D), jnp.float32),
              jax.ShapeDtypeStruct((NC, NP // 128, 128), jnp.float32)),
    mesh=_mesh,
    compiler_params=pltpu.CompilerParams(needs_layout_passes=False),
    scratch_types=[
        pltpu.VMEM((2, 1, BLK), jnp.int32),
        pltpu.VMEM((2, 1, BLK), jnp.int32),
        pltpu.VMEM((2, 1, BLK), jnp.float32),
        pltpu.VMEM((2, BLK, D), jnp.float32),
        pltpu.VMEM((BLK,), jnp.float32),
        pltpu.VMEM((NP,), jnp.float32),
        pltpu.VMEM((NP // 128, 128), jnp.float32),
        pltpu.VMEM((1, NP // 128), jnp.int32),
        pltpu.VMEM((PS,), jnp.float32),
        pltpu.VMEM((PS,), jnp.float32),
        pltpu.VMEM_SHARED((NP, D), jnp.float32),
        pltpu.VMEM_SHARED((NP // 128, 128), jnp.float32),
        pltpu.SemaphoreType.DMA,
        pltpu.SemaphoreType.DMA,
        pltpu.SemaphoreType.DMA,
        pltpu.SemaphoreType.DMA,
    ],
)
def _sc_aggregate(v_hbm, alpha_hbm, m2_hbm, dst_hbm, src_hbm,
                  out2_hbm, den2_hbm,
                  dsti, srci, abufb, vrows, wbuf, mloc, dpriv, idx80,
                  redbuf, redbuf2, oshared, dshared, i0, i1, g0, g1):
    sid = lax.axis_index("s")
    cid = lax.axis_index("c")
    w = _wid()
    isem = (i0, i1)
    gsem = (g0, g1)

    def istart(b, par):
        pltpu.async_copy(dst_hbm.at[w, pl.ds(b, 1)], dsti.at[par],
                         isem[par])
        pltpu.async_copy(src_hbm.at[w, pl.ds(b, 1)], srci.at[par],
                         isem[par])
        pltpu.async_copy(alpha_hbm.at[w, pl.ds(b, 1)], abufb.at[par],
                         isem[par])

    def iwait(par):
        pltpu.make_async_copy(dst_hbm.at[w, pl.ds(0, 1)], dsti.at[par],
                              isem[par]).wait()
        pltpu.make_async_copy(src_hbm.at[w, pl.ds(0, 1)], srci.at[par],
                              isem[par]).wait()
        pltpu.make_async_copy(alpha_hbm.at[w, pl.ds(0, 1)], abufb.at[par],
                              isem[par]).wait()

    def gstart(b, par):
        pltpu.async_copy(v_hbm.at[srci.at[par, 0]], vrows.at[par],
                         gsem[par])

    def gwait(par):
        pltpu.make_async_copy(v_hbm.at[srci.at[0, 0]], vrows.at[par],
                              gsem[par]).wait()

    istart(0, 0)
    istart(1, 1)

    def mker(i, c):
        s = pl.ds(i * PS, PS)
        pltpu.sync_copy(m2_hbm.at[0, s], redbuf)
        pltpu.sync_copy(m2_hbm.at[1, s], redbuf2)

        def mx(j, c2):
            ss = pl.ds(j * LANES, LANES)
            mloc[pl.ds(i * PS + j * LANES, LANES)] = jnp.maximum(
                redbuf[ss], redbuf2[ss])
            return c2

        lax.fori_loop(0, PS // LANES, mx, 0)
        return c

    lax.fori_loop(0, NP // PS, mker, 0)

    iota = lax.iota(jnp.int32, LANES)

    def zd(r, c):
        for j in range(D // LANES):
            dpriv[r, pl.ds(j * LANES, LANES)] = jnp.zeros((LANES,),
                                                          jnp.float32)
        return c

    lax.fori_loop(0, NP // 128, zd, 0)
    for j in range(NP // 128 // LANES):
        idx80[0, pl.ds(j * LANES, LANES)] = iota + j * LANES

    def zv(r, c):
        for j in range(D // LANES):
            vrows[0, r, pl.ds(j * LANES, LANES)] = jnp.zeros(
                (LANES,), jnp.float32)
        return c

    lax.fori_loop(0, BLK, zv, 0)

    def zo(i, c):
        pltpu.sync_copy(vrows.at[0],
                        oshared.at[pl.ds(sid * PS + i * BLK, BLK)])
        return c

    lax.fori_loop(0, PS // BLK, zo, 0)
    @pl.when(sid < NP // 128 // 8)
    def _():
        pltpu.sync_copy(vrows.at[0, pl.ds(0, 8)],
                        dshared.at[pl.ds(sid * 8, 8)])

    plsc.subcore_barrier()
    iwait(0)
    gstart(0, 0)

    def blk(b, par):
        @pl.when(b + 1 < NBLK)
        def _():
            iwait(1 - par)
            gstart(b + 1, 1 - par)

        gwait(par)
        for g in range(GPB):
            ss = pl.ds(g * LANES, LANES)
            dg = dsti[par, 0, ss]
            mg = plsc.load_gather(mloc, [dg])
            ag = jnp.exp(abufb[par, 0, ss] - mg)
            wbuf[ss] = ag
            dr = lax.shift_right_logical(dg, 7)
            dc = jnp.bitwise_and(dg, 127)
            plsc.addupdate_scatter(dpriv, [dr, dc], ag)

        def scale(e, c2):
            asp = plsc.load_gather(wbuf,
                                   [jnp.full((LANES,), 0, jnp.int32) + e])
            for j in range(D // LANES):
                s2 = pl.ds(j * LANES, LANES)
                vrows[par, e, s2] = vrows[par, e, s2] * asp
            return c2

        lax.fori_loop(0, BLK, scale, 0)
        pltpu.sync_copy(vrows.at[par], oshared.at[dsti.at[par, 0]],
                        add=True)

        @pl.when(b + 2 < NBLK)
        def _():
            istart(b + 2, par)

    def pair(s, c):
        blk(2 * s, 0)
        blk(2 * s + 1, 1)
        return c

    lax.fori_loop(0, (NBLK - 1) // 2, pair, 0)
    blk(NBLK - 1, 0)

    pltpu.sync_copy(dpriv, dshared.at[idx80.at[0]], add=True)
    plsc.subcore_barrier()
    nbase = sid * PS

    @pl.when(sid < NP // 128 // 8)
    def _():
        pltpu.sync_copy(dshared.at[pl.ds(sid * 8, 8)],
                        den2_hbm.at[cid, pl.ds(sid * 8, 8)])

    pltpu.sync_copy(oshared.at[pl.ds(nbase, PS)],
                    out2_hbm.at[cid, pl.ds(nbase, PS)])


BS = 400
GR = N // BS
EB = E // GR


def _tc_pre_body(x_ref, w_ref, b_ref, ea_ref, wb_ref,
                 q_ref, k_ref, v_ref, s_ref, eat_ref):
    y = jnp.dot(x_ref[...], w_ref[...],
                preferred_element_type=jnp.float32) + b_ref[...]
    q_ref[...] = y[:, :D]
    k_ref[...] = y[:, D:2 * D]
    v_ref[...] = y[:, 2 * D:3 * D]
    s_ref[...] = y[:, 3 * D:]
    eat_ref[...] = ea_ref[...] * wb_ref[0, 0] + wb_ref[0, 1]


def _eb_spec():
    return pl.BlockSpec((1, 1, EB), lambda i: (i, 0, 0))


def _tc_pre(h, wcat, bcat, ea2d, wb):
    return pl.pallas_call(
        _tc_pre_body,
        grid=(GR,),
        in_specs=[pl.BlockSpec((BS, D), lambda i: (i, 0)),
                  pl.BlockSpec((D, 4 * D), lambda i: (0, 0)),
                  pl.BlockSpec((1, 4 * D), lambda i: (0, 0)),
                  _eb_spec(),
                  pl.BlockSpec((1, 128), lambda i: (0, 0))],
        out_specs=[pl.BlockSpec((BS, D), lambda i: (i, 0))] * 4 +
                  [_eb_spec()],
        out_shape=[jax.ShapeDtypeStruct((N, D), jnp.float32)] * 4 +
                  [jax.ShapeDtypeStruct((GR, 1, EB), jnp.float32)],
    )(h, wcat, bcat, ea2d, wb)


def _tc_post_body(o2_ref, d2_ref, s_ref, g1_ref, b1_ref, g2_ref, b2_ref,
                  h_ref):
    o = o2_ref[0, :N, :] + o2_ref[1, :N, :]
    den = d2_ref[0, :N, :] + d2_ref[1, :N, :]
    x = o / (den + 1e-16) + s_ref[...]
    mu = jnp.mean(x, axis=0, keepdims=True)
    xc = x - mu
    var = jnp.mean(xc * xc, axis=0, keepdims=True)
    x = xc / jnp.sqrt(var + 1e-5) * g1_ref[...] + b1_ref[...]
    mu = jnp.mean(x, axis=0, keepdims=True)
    xc = x - mu
    var = jnp.mean(xc * xc, axis=0, keepdims=True)
    x = xc / jnp.sqrt(var + 1e-5) * g2_ref[...] + b2_ref[...]
    h_ref[...] = jax.nn.softplus(x)


def _tc_post(out2, den2, s, g1, b1, g2, b2):
    return pl.pallas_call(
        _tc_post_body,
        out_shape=jax.ShapeDtypeStruct((N, D), jnp.float32),
    )(out2, den2, s, g1, b1, g2, b2)


def _tc_final_body(h_ref, w1t_ref, b1_ref, w2_ref, b2_ref, o_ref):
    pooled = jnp.mean(h_ref[...], axis=0, keepdims=True)
    hid = jax.nn.softplus(
        jnp.dot(pooled, w1t_ref[...],
                preferred_element_type=jnp.float32) + b1_ref[...])
    pred = jnp.sum(hid * w2_ref[...]) + b2_ref[0, 0]
    o_ref[...] = jnp.zeros((8, 128), jnp.float32) + pred


def _tc_final(h, w1t, b1, w2, b2):
    return pl.pallas_call(
        _tc_final_body,
        out_shape=jax.ShapeDtypeStruct((8, 128), jnp.float32),
    )(h, w1t, b1, w2, b2)


def kernel(atomic_numbers, edge_index, edge_attr, embed_table, Wq, bq, Wk,
           bk, Wv, bv, We, be, Ws, bs, bn1_g, bn1_b, bn2_g, bn2_b, W1, b1,
           W2, b2):
    anum = jnp.pad(atomic_numbers.astype(jnp.int32), (0, NP - N))
    src = edge_index[0].astype(jnp.int32).reshape(NW, NBLK, BLK)
    dst = edge_index[1].astype(jnp.int32).reshape(NW, NBLK, BLK)
    ea2d = edge_attr[:, 0].astype(jnp.float32).reshape(GR, 1, EB)
    h = _sc_embed(anum, embed_table)[:N]
    for l in range(L):
        wcat = jnp.concatenate([Wq[l], Wk[l], Wv[l], Ws[l]], axis=0).T
        bcat = jnp.concatenate([bq[l], bk[l], bv[l], bs[l]]).reshape(
            1, 4 * D)
        wb = jnp.zeros((1, 128), jnp.float32).at[0, 0].set(
            We[l, 0, 0]).at[0, 1].set(be[l, 0])
        q, k, v, s, eat = _tc_pre(h, wcat, bcat, ea2d, wb)
        alpha, m2 = _sc_alpha(q, k, dst, src, eat.reshape(NW, NBLK, BLK))
        out2, den2 = _sc_aggregate(v, alpha, m2, dst, src)
        h = _tc_post(out2, den2.reshape(NC, NP, 1), s,
                     bn1_g[l].reshape(1, D), bn1_b[l].reshape(1, D),
                     bn2_g[l].reshape(1, D), bn2_b[l].reshape(1, D))
    out = _tc_final(h, W1.T, b1.reshape(1, 2 * D), W2, b2.reshape(1, 1))
    return out[0, :1]


# speedup vs baseline: 9.1395x; 2.2265x over previous
import functools

import jax
import jax.numpy as jnp
import numpy as np
from jax import lax
from jax.experimental import pallas as pl
from jax.experimental.pallas import tpu as pltpu
from jax.experimental.pallas import tpu_sc as plsc

N = 10000
E = 320000
D = 128
L = 3
NC = 2
NS = 16
NW = NC * NS
LANES = 16
NP = 10240
PS = NP // NS
EPW = E // NW
BLK = 80
NBLK = EPW // BLK
GPB = BLK // LANES

_mesh = plsc.VectorSubcoreMesh(core_axis_name="c", subcore_axis_name="s")


def _wid():
    return lax.axis_index("s") * NC + lax.axis_index("c")


@functools.partial(
    pl.kernel,
    out_type=jax.ShapeDtypeStruct((NP, D), jnp.float32),
    mesh=_mesh,
    compiler_params=pltpu.CompilerParams(needs_layout_passes=False),
    scratch_types=[
        pltpu.VMEM((1, BLK), jnp.int32),
        pltpu.VMEM((BLK, D), jnp.float32),
        pltpu.SemaphoreType.DMA,
    ],
)
def _sc_embed(anum_hbm, table_hbm, h_hbm, idxb, rows, sem):
    w = _wid()
    base = w * (NP // NW)

    def body(i, c):
        off = base + i * BLK
        pltpu.sync_copy(anum_hbm.at[pl.ds(off, BLK)], idxb.at[0])
        pltpu.async_copy(table_hbm.at[idxb.at[0]], rows, sem).wait()
        pltpu.sync_copy(rows, h_hbm.at[pl.ds(off, BLK)])
        return c

    lax.fori_loop(0, (NP // NW) // BLK, body, 0)


@functools.partial(
    pl.kernel,
    out_type=(jax.ShapeDtypeStruct((E, D), jnp.float32),
              jax.ShapeDtypeStruct((E, D), jnp.float32)),
    mesh=_mesh,
    compiler_params=pltpu.CompilerParams(needs_layout_passes=False),
    scratch_types=[
        pltpu.VMEM((NBLK, BLK), jnp.int32),
        pltpu.VMEM((NBLK, BLK), jnp.int32),
        pltpu.VMEM((3, BLK, D), jnp.float32),
        pltpu.VMEM((3, BLK, D), jnp.float32),
        pltpu.SemaphoreType.DMA,
        pltpu.SemaphoreType.DMA,
        pltpu.SemaphoreType.DMA,
        pltpu.SemaphoreType.DMA,
        pltpu.SemaphoreType.DMA,
        pltpu.SemaphoreType.DMA,
    ],
)
def _sc_gatherqk(q_hbm, k_hbm, dst_hbm, src_hbm, qg_hbm, kg_hbm,
                 dsti, srci, qrows, krows, g0, g1, g2, w0, w1, w2):
    w = _wid()
    ebase = w * EPW
    gsem = (g0, g1, g2)
    wsem = (w0, w1, w2)

    pltpu.sync_copy(dst_hbm.at[w], dsti)
    pltpu.sync_copy(src_hbm.at[w], srci)

    def gstart(b, p):
        pltpu.async_copy(q_hbm.at[dsti.at[b]], qrows.at[p], gsem[p])
        pltpu.async_copy(k_hbm.at[srci.at[b]], krows.at[p], gsem[p])

    def gwait(p):
        pltpu.make_async_copy(q_hbm.at[dsti.at[0]], qrows.at[p],
                              gsem[p]).wait()
        pltpu.make_async_copy(k_hbm.at[srci.at[0]], krows.at[p],
                              gsem[p]).wait()

    def wstart(b, p):
        off = ebase + b * BLK
        pltpu.async_copy(qrows.at[p], qg_hbm.at[pl.ds(off, BLK)], wsem[p])
        pltpu.async_copy(krows.at[p], kg_hbm.at[pl.ds(off, BLK)], wsem[p])

    def wwait(p):
        pltpu.make_async_copy(qrows.at[p], qg_hbm.at[pl.ds(0, BLK)],
                              wsem[p]).wait()
        pltpu.make_async_copy(krows.at[p], kg_hbm.at[pl.ds(0, BLK)],
                              wsem[p]).wait()

    gstart(0, 0)
    gstart(1, 1)

    def triple(s, c):
        for t in range(3):
            b = 3 * s + t
            gwait(t)
            if t == 0:
                @pl.when(s > 0)
                def _():
                    wwait(2)
            else:
                wwait(t - 1)
            wstart(b, t)

            @pl.when(b + 2 < NBLK)
            def _():
                gstart(b + 2, (t + 2) % 3)
        return c

    lax.fori_loop(0, NBLK // 3, triple, 0)
    gwait(0)
    wwait(2)
    wstart(NBLK - 2, 0)
    gwait(1)
    wwait(0)
    wstart(NBLK - 1, 1)
    wwait(1)


@functools.partial(
    pl.kernel,
    out_type=jax.ShapeDtypeStruct((NC, NP), jnp.float32),
    mesh=_mesh,
    compiler_params=pltpu.CompilerParams(needs_layout_passes=False),
    scratch_types=[
        pltpu.VMEM((NBLK, BLK), jnp.int32),
        pltpu.VMEM((NBLK, BLK), jnp.float32),
        pltpu.VMEM((NP,), jnp.float32),
        pltpu.VMEM((PS,), jnp.float32),
        pltpu.VMEM((PS,), jnp.float32),
        pltpu.VMEM_SHARED((NS, NP), jnp.float32),
    ],
)
def _sc_max(alpha_hbm, dst_hbm, m2_hbm,
            dsti, av, mpriv, redbuf, redbuf2, mshared):
    sid = lax.axis_index("s")
    cid = lax.axis_index("c")
    w = _wid()
    pltpu.sync_copy(dst_hbm.at[w], dsti)
    pltpu.sync_copy(alpha_hbm.at[w], av)

    def initb(i, c):
        mpriv[pl.ds(i * LANES, LANES)] = jnp.full((LANES,), -jnp.inf,
                                                  jnp.float32)
        return c

    lax.fori_loop(0, NP // LANES, initb, 0)

    def blk(b, c):
        for g in range(GPB):
            ss = pl.ds(g * LANES, LANES)
            ag = av[b, ss]
            dg = dsti[b, ss]
            cur = plsc.load_gather(mpriv, [dg])
            act = jnp.where(ag > cur, jnp.int32(1), jnp.int32(0))

            def rcond(a):
                return jnp.max(a) > 0

            def rbody(a):
                msk = a > 0
                plsc.store_scatter(mpriv, [dg], ag, mask=msk)
                c2 = plsc.load_gather(mpriv, [dg])
                return jnp.where(msk & (ag > c2), jnp.int32(1),
                                 jnp.int32(0))

            lax.while_loop(rcond, rbody, act)
        return c

    lax.fori_loop(0, NBLK, blk, 0)

    pltpu.sync_copy(mpriv, mshared.at[sid])
    plsc.subcore_barrier()
    nbase = sid * PS
    pltpu.sync_copy(mshared.at[0, pl.ds(nbase, PS)], redbuf)

    def tred(t, c):
        pltpu.sync_copy(mshared.at[t, pl.ds(nbase, PS)], redbuf2)

        def mx(i, c2):
            s = pl.ds(i * LANES, LANES)
            redbuf[s] = jnp.maximum(redbuf[s], redbuf2[s])
            return c2

        lax.fori_loop(0, PS // LANES, mx, 0)
        return c

    lax.fori_loop(1, NS, tred, 0)
    pltpu.sync_copy(redbuf, m2_hbm.at[cid, pl.ds(nbase, PS)])


@functools.partial(
    pl.kernel,
    out_type=(jax.ShapeDtypeStruct((NC, NP, D), jnp.float32),
              jax.ShapeDtypeStruct((NC, NP // 128, 128), jnp.float32)),
    mesh=_mesh,
    compiler_params=pltpu.CompilerParams(needs_layout_passes=False),
    scratch_types=[
        pltpu.VMEM((2, 1, BLK), jnp.int32),
        pltpu.VMEM((2, 1, BLK), jnp.int32),
        pltpu.VMEM((2, 1, BLK), jnp.float32),
        pltpu.VMEM((2, BLK, D), jnp.float32),
        pltpu.VMEM((BLK,), jnp.float32),
        pltpu.VMEM((NP,), jnp.float32),
        pltpu.VMEM((NP // 128, 128), jnp.float32),
        pltpu.VMEM((1, NP // 128), jnp.int32),
        pltpu.VMEM((PS,), jnp.float32),
        pltpu.VMEM((PS,), jnp.float32),
        pltpu.VMEM_SHARED((NP, D), jnp.float32),
        pltpu.VMEM_SHARED((NP // 128, 128), jnp.float32),
        pltpu.SemaphoreType.DMA,
        pltpu.SemaphoreType.DMA,
        pltpu.SemaphoreType.DMA,
        pltpu.SemaphoreType.DMA,
    ],
)
def _sc_aggregate(v_hbm, alpha_hbm, m2_hbm, dst_hbm, src_hbm,
                  out2_hbm, den2_hbm,
                  dsti, srci, abufb, vrows, wbuf, mloc, dpriv, idx80,
                  redbuf, redbuf2, oshared, dshared, i0, i1, g0, g1):
    sid = lax.axis_index("s")
    cid = lax.axis_index("c")
    w = _wid()
    isem = (i0, i1)
    gsem = (g0, g1)

    def istart(b, par):
        pltpu.async_copy(dst_hbm.at[w, pl.ds(b, 1)], dsti.at[par],
                         isem[par])
        pltpu.async_copy(src_hbm.at[w, pl.ds(b, 1)], srci.at[par],
                         isem[par])
        pltpu.async_copy(alpha_hbm.at[w, pl.ds(b, 1)], abufb.at[par],
                         isem[par])

    def iwait(par):
        pltpu.make_async_copy(dst_hbm.at[w, pl.ds(0, 1)], dsti.at[par],
                              isem[par]).wait()
        pltpu.make_async_copy(src_hbm.at[w, pl.ds(0, 1)], srci.at[par],
                              isem[par]).wait()
        pltpu.make_async_copy(alpha_hbm.at[w, pl.ds(0, 1)], abufb.at[par],
                              isem[par]).wait()

    def gstart(b, par):
        pltpu.async_copy(v_hbm.at[srci.at[par, 0]], vrows.at[par],
                         gsem[par])

    def gwait(par):
        pltpu.make_async_copy(v_hbm.at[srci.at[0, 0]], vrows.at[par],
                              gsem[par]).wait()

    istart(0, 0)
    istart(1, 1)

    def mker(i, c):
        s = pl.ds(i * PS, PS)
        pltpu.sync_copy(m2_hbm.at[0, s], redbuf)
        pltpu.sync_copy(m2_hbm.at[1, s], redbuf2)

        def mx(j, c2):
            ss = pl.ds(j * LANES, LANES)
            mloc[pl.ds(i * PS + j * LANES, LANES)] = jnp.maximum(
                redbuf[ss], redbuf2[ss])
            return c2

        lax.fori_loop(0, PS // LANES, mx, 0)
        return c

    lax.fori_loop(0, NP // PS, mker, 0)

    iota = lax.iota(jnp.int32, LANES)

    def zd(r, c):
        for j in range(D // LANES):
            dpriv[r, pl.ds(j * LANES, LANES)] = jnp.zeros((LANES,),
                                                          jnp.float32)
        return c

    lax.fori_loop(0, NP // 128, zd, 0)
    for j in range(NP // 128 // LANES):
        idx80[0, pl.ds(j * LANES, LANES)] = iota + j * LANES

    def zv(r, c):
        for j in range(D // LANES):
            vrows[0, r, pl.ds(j * LANES, LANES)] = jnp.zeros(
                (LANES,), jnp.float32)
        return c

    lax.fori_loop(0, BLK, zv, 0)

    def zo(i, c):
        pltpu.sync_copy(vrows.at[0],
                        oshared.at[pl.ds(sid * PS + i * BLK, BLK)])
        return c

    lax.fori_loop(0, PS // BLK, zo, 0)
    @pl.when(sid < NP // 128 // 8)
    def _():
        pltpu.sync_copy(vrows.at[0, pl.ds(0, 8)],
                        dshared.at[pl.ds(sid * 8, 8)])

    plsc.subcore_barrier()
    iwait(0)
    gstart(0, 0)

    def blk(b, par):
        @pl.when(b + 1 < NBLK)
        def _():
            iwait(1 - par)
            gstart(b + 1, 1 - par)

        gwait(par)
        for g in range(GPB):
            ss = pl.ds(g * LANES, LANES)
            dg = dsti[par, 0, ss]
            mg = plsc.load_gather(mloc, [dg])
            ag = jnp.exp(abufb[par, 0, ss] - mg)
            wbuf[ss] = ag
            dr = lax.shift_right_logical(dg, 7)
            dc = jnp.bitwise_and(dg, 127)
            plsc.addupdate_scatter(dpriv, [dr, dc], ag)

        def scale(e, c2):
            asp = plsc.load_gather(wbuf,
                                   [jnp.full((LANES,), 0, jnp.int32) + e])
            for j in range(D // LANES):
                s2 = pl.ds(j * LANES, LANES)
                vrows[par, e, s2] = vrows[par, e, s2] * asp
            return c2

        lax.fori_loop(0, BLK, scale, 0)
        pltpu.sync_copy(vrows.at[par], oshared.at[dsti.at[par, 0]],
                        add=True)

        @pl.when(b + 2 < NBLK)
        def _():
            istart(b + 2, par)

    def pair(s, c):
        blk(2 * s, 0)
        blk(2 * s + 1, 1)
        return c

    lax.fori_loop(0, (NBLK - 1) // 2, pair, 0)
    blk(NBLK - 1, 0)

    pltpu.sync_copy(dpriv, dshared.at[idx80.at[0]], add=True)
    plsc.subcore_barrier()
    nbase = sid * PS

    @pl.when(sid < NP // 128 // 8)
    def _():
        pltpu.sync_copy(dshared.at[pl.ds(sid * 8, 8)],
                        den2_hbm.at[cid, pl.ds(sid * 8, 8)])

    pltpu.sync_copy(oshared.at[pl.ds(nbase, PS)],
                    out2_hbm.at[cid, pl.ds(nbase, PS)])


BS = 400
GR = N // BS
EB = E // GR


def _tc_pre_body(x_ref, w_ref, b_ref, ea_ref, wb_ref,
                 q_ref, k_ref, v_ref, s_ref, eat_ref):
    y = jnp.dot(x_ref[...], w_ref[...],
                preferred_element_type=jnp.float32) + b_ref[...]
    q_ref[...] = y[:, :D]
    k_ref[...] = y[:, D:2 * D]
    v_ref[...] = y[:, 2 * D:3 * D]
    s_ref[...] = y[:, 3 * D:]
    eat_ref[...] = ea_ref[...] * wb_ref[0, 0] + wb_ref[0, 1]


def _eb_spec():
    return pl.BlockSpec((1, 1, EB), lambda i: (i, 0, 0))


def _tc_pre(h, wcat, bcat, ea2d, wb):
    return pl.pallas_call(
        _tc_pre_body,
        grid=(GR,),
        in_specs=[pl.BlockSpec((BS, D), lambda i: (i, 0)),
                  pl.BlockSpec((D, 4 * D), lambda i: (0, 0)),
                  pl.BlockSpec((1, 4 * D), lambda i: (0, 0)),
                  _eb_spec(),
                  pl.BlockSpec((1, 128), lambda i: (0, 0))],
        out_specs=[pl.BlockSpec((BS, D), lambda i: (i, 0))] * 4 +
                  [_eb_spec()],
        out_shape=[jax.ShapeDtypeStruct((N, D), jnp.float32)] * 4 +
                  [jax.ShapeDtypeStruct((GR, 1, EB), jnp.float32)],
    )(h, wcat, bcat, ea2d, wb)


def _tc_dot_body(qg_ref, kg_ref, eat_ref, a_ref):
    prod = qg_ref[...] * kg_ref[...]
    ones = jnp.ones((1, D), jnp.float32)
    s = jax.lax.dot_general(ones, prod, (((1,), (1,)), ((), ())),
                            preferred_element_type=jnp.float32)
    a_ref[...] = (s * jnp.float32(1.0 / np.sqrt(D))).reshape(1, 1, EB) \
        + eat_ref[...]


def _tc_dot(qg, kg, eat):
    return pl.pallas_call(
        _tc_dot_body,
        grid=(GR,),
        in_specs=[pl.BlockSpec((EB, D), lambda i: (i, 0)),
                  pl.BlockSpec((EB, D), lambda i: (i, 0)),
                  _eb_spec()],
        out_specs=_eb_spec(),
        out_shape=jax.ShapeDtypeStruct((GR, 1, EB), jnp.float32),
    )(qg, kg, eat)


def _tc_post_body(o2_ref, d2_ref, s_ref, g1_ref, b1_ref, g2_ref, b2_ref,
                  h_ref):
    o = o2_ref[0, :N, :] + o2_ref[1, :N, :]
    den = d2_ref[0, :N, :] + d2_ref[1, :N, :]
    x = o / (den + 1e-16) + s_ref[...]
    mu = jnp.mean(x, axis=0, keepdims=True)
    xc = x - mu
    var = jnp.mean(xc * xc, axis=0, keepdims=True)
    x = xc / jnp.sqrt(var + 1e-5) * g1_ref[...] + b1_ref[...]
    mu = jnp.mean(x, axis=0, keepdims=True)
    xc = x - mu
    var = jnp.mean(xc * xc, axis=0, keepdims=True)
    x = xc / jnp.sqrt(var + 1e-5) * g2_ref[...] + b2_ref[...]
    h_ref[...] = jax.nn.softplus(x)


def _tc_post(out2, den2, s, g1, b1, g2, b2):
    return pl.pallas_call(
        _tc_post_body,
        out_shape=jax.ShapeDtypeStruct((N, D), jnp.float32),
    )(out2, den2, s, g1, b1, g2, b2)


def _tc_final_body(h_ref, w1t_ref, b1_ref, w2_ref, b2_ref, o_ref):
    pooled = jnp.mean(h_ref[...], axis=0, keepdims=True)
    hid = jax.nn.softplus(
        jnp.dot(pooled, w1t_ref[...],
                preferred_element_type=jnp.float32) + b1_ref[...])
    pred = jnp.sum(hid * w2_ref[...]) + b2_ref[0, 0]
    o_ref[...] = jnp.zeros((8, 128), jnp.float32) + pred


def _tc_final(h, w1t, b1, w2, b2):
    return pl.pallas_call(
        _tc_final_body,
        out_shape=jax.ShapeDtypeStruct((8, 128), jnp.float32),
    )(h, w1t, b1, w2, b2)


def kernel(atomic_numbers, edge_index, edge_attr, embed_table, Wq, bq, Wk,
           bk, Wv, bv, We, be, Ws, bs, bn1_g, bn1_b, bn2_g, bn2_b, W1, b1,
           W2, b2):
    anum = jnp.pad(atomic_numbers.astype(jnp.int32), (0, NP - N))
    src = edge_index[0].astype(jnp.int32).reshape(NW, NBLK, BLK)
    dst = edge_index[1].astype(jnp.int32).reshape(NW, NBLK, BLK)
    ea2d = edge_attr[:, 0].astype(jnp.float32).reshape(GR, 1, EB)
    h = _sc_embed(anum, embed_table)[:N]
    for l in range(L):
        wcat = jnp.concatenate([Wq[l], Wk[l], Wv[l], Ws[l]], axis=0).T
        bcat = jnp.concatenate([bq[l], bk[l], bv[l], bs[l]]).reshape(
            1, 4 * D)
        wb = jnp.zeros((1, 128), jnp.float32).at[0, 0].set(
            We[l, 0, 0]).at[0, 1].set(be[l, 0])
        q, k, v, s, eat = _tc_pre(h, wcat, bcat, ea2d, wb)
        qg, kg = _sc_gatherqk(q, k, dst, src)
        alpha = _tc_dot(qg, kg, eat).reshape(NW, NBLK, BLK)
        m2 = _sc_max(alpha, dst)
        out2, den2 = _sc_aggregate(v, alpha, m2, dst, src)
        h = _tc_post(out2, den2.reshape(NC, NP, 1), s,
                     bn1_g[l].reshape(1, D), bn1_b[l].reshape(1, D),
                     bn2_g[l].reshape(1, D), bn2_b[l].reshape(1, D))
    out = _tc_final(h, W1.T, b1.reshape(1, 2 * D), W2, b2.reshape(1, 1))
    return out[0, :1]
